# baseline (speedup 1.0000x reference)
import jax
import jax.numpy as jnp
from jax import lax
from jax.experimental import pallas as pl
from jax.experimental.pallas import tpu as pltpu
from jax.experimental.pallas import tpu_sc as plsc

N = 5000
NT = 16
C = 320
NP = NT * C
NV = C // 16
K = 8
KB = K.bit_length() - 1
TCW = NT * K
DETS = 100
NMS_THRESH = 0.5
SCORE_THRESH = 0.05
FNEG = -3.4e38
IBIG = 2**31 - 1


def _nms_body(bx_hbm, sc_hbm, out_hbm, boxes_v, area_v, live_v, fill_v,
              rec_v, recs_v, out_v, recs_s):
    tid = lax.axis_index("s")
    base = tid * C
    io = lax.iota(jnp.int32, 16)

    for c in range(4):
        pltpu.sync_copy(bx_hbm.at[pl.ds(c * NP + base, C)], boxes_v.at[c])
    pltpu.sync_copy(sc_hbm.at[pl.ds(base, C)], live_v)
    pltpu.sync_copy(sc_hbm.at[pl.ds(base, C)], fill_v)

    for j in range(NV):
        sl = pl.ds(j * 16, 16)
        area_v[sl] = ((boxes_v[2, sl] - boxes_v[0, sl]) *
                      (boxes_v[3, sl] - boxes_v[1, sl]))

    def _tree(op, xs):
        xs = list(xs)
        while len(xs) > 1:
            xs = [op(xs[i], xs[i + 1]) if i + 1 < len(xs) else xs[i]
                  for i in range(0, len(xs), 2)]
        return xs[0]

    def topk_insert(v, g, st):
        cs = [v > m for m, _ in st]
        out = []
        for k in range(K):
            mk, ik = st[k]
            if k == 0:
                out.append((jnp.where(cs[0], v, mk), jnp.where(cs[0], g, ik)))
            else:
                mp, ip = st[k - 1]
                out.append((jnp.where(cs[k - 1], mp, jnp.where(cs[k], v, mk)),
                            jnp.where(cs[k - 1], ip, jnp.where(cs[k], g, ik))))
        return tuple(out)

    def topk_init():
        return tuple((jnp.full((16,), FNEG, jnp.float32),
                      jnp.full((16,), 0, jnp.int32)) for _ in range(K))

    def init_pass():
        st = topk_init()
        for j in range(NV):
            sl = pl.ds(j * 16, 16)
            st = topk_insert(live_v[sl], base + j * 16 + io, st)
        return st

    def tile_extract(st):
        vals = [v for v, _ in st]
        idxs = [i for _, i in st]
        out = []
        for _ in range(K):
            gm = jnp.max(_tree(jnp.maximum, vals))
            gi = jnp.min(_tree(jnp.minimum,
                               [jnp.where(vals[k] == gm, idxs[k], IBIG)
                                for k in range(K)]))
            out.append((gm, gi))
            vals = [jnp.where(idxs[k] == gi, FNEG, vals[k]) for k in range(K)]
        return out

    def make_rec(s, gidx):
        lidx = jnp.clip(gidx - base, 0, C - 1)
        cvec = plsc.load_gather(boxes_v, [jnp.clip(io - 2, 0, 3),
                                          jnp.full((16,), lidx, jnp.int32)])
        avec = plsc.load_gather(area_v, [jnp.full((16,), lidx, jnp.int32)])
        return jnp.where(io == 0, s,
               jnp.where(io == 1, gidx.astype(jnp.float32),
               jnp.where(io == 6, avec, cvec)))

    def publish(slot, cands):
        for e, (s, gi) in enumerate(cands):
            rec_v[pl.ds(e * 16, 16)] = make_rec(s, gi)
        pltpu.sync_copy(rec_v, recs_s.at[slot, tid])
        plsc.subcore_barrier()
        pltpu.sync_copy(recs_s.at[slot], recs_v)

    def col(c, f):
        return plsc.load_gather(recs_v, [io, jnp.full((16,), c * 16 + f,
                                                      jnp.int32)])

    def splat_field(t_e, off_e):
        return plsc.load_gather(recs_v, [jnp.full((16,), t_e, jnp.int32),
                                         jnp.full((16,), off_e, jnp.int32)])

    def global_extract():
        sv = [col(c, 0) for c in range(K)]
        ivi = [col(c, 1).astype(jnp.int32) for c in range(K)]
        winners = []
        for _ in range(K):
            gm = jnp.max(_tree(jnp.maximum, sv))
            key = jnp.min(_tree(jnp.minimum,
                                [jnp.where(sv[c] == gm,
                                           ivi[c] * TCW + io * K + c, IBIG)
                                 for c in range(K)]))
            gi = lax.shift_right_logical(key, 4 + KB)
            tc = key & (TCW - 1)
            winners.append((gm, gi, tc))
            sv = [jnp.where(ivi[c] == gi, FNEG, sv[c]) for c in range(K)]
        return winners

    def fetch_coords(tc):
        t_e = lax.shift_right_logical(tc, KB)
        o_e = (tc & (K - 1)) * 16
        return (splat_field(t_e, o_e + 2), splat_field(t_e, o_e + 3),
                splat_field(t_e, o_e + 4), splat_field(t_e, o_e + 5),
                splat_field(t_e, o_e + 6))

    def overlap_vec(ax1, ay1, ax2, ay2, aar, bx1, by1, bx2, by2, bar):
        w = jnp.maximum(jnp.minimum(ax2, bx2) - jnp.maximum(ax1, bx1), 0.0)
        h = jnp.maximum(jnp.minimum(ay2, by2) - jnp.maximum(ay1, by1), 0.0)
        inter = w * h
        union = (aar + bar) - inter
        return (inter + inter) > union

    def emit_row(pos_vec, x1, y1, x2, y2, s, mask):
        row = jnp.where(io == 0, x1,
              jnp.where(io == 1, y1,
              jnp.where(io == 2, x2,
              jnp.where(io == 3, y2,
              jnp.where(io == 4, s, 0.0)))))
        posc = jnp.minimum(pos_vec, DETS - 1)
        @pl.when(tid == 0)
        def _():
            plsc.store_scatter(out_v, [posc * 16 + io], row, mask=mask)

    def mark_emitted(gidx, keep_mask):
        lidx = gidx.astype(jnp.int32) - base
        inr = (lidx >= 0) & (lidx < C)
        idxv = jnp.full((16,), jnp.clip(lidx, 0, C - 1), jnp.int32)
        plsc.store_scatter(fill_v, [idxv], jnp.full((16,), -2.0, jnp.float32),
                           mask=(io == 0) & inr & keep_mask)

    st0 = init_pass()

    def round_body(carry):
        cnt, rnd, st_flat = carry
        st = tuple((st_flat[2 * k], st_flat[2 * k + 1]) for k in range(K))
        slot = lax.rem(rnd, 2)
        publish(slot, tile_extract(st))
        winners = global_extract()
        keepable = winners[0][0] > SCORE_THRESH

        def keep_branch(_):
            coords = [fetch_coords(tc) for (_, _, tc) in winners]
            kept = []
            for e in range(K):
                s_e = winners[e][0]
                ok = jnp.full((16,), True)
                for i in range(e):
                    ov = overlap_vec(*coords[i], *coords[e])
                    ok = ok & ~(kept[i] & ov)
                kept.append(ok & (s_e > SCORE_THRESH))
            pos = jnp.full((16,), 0, jnp.int32) + cnt
            for e in range(K):
                x1, y1, x2, y2, _ = coords[e]
                emit_row(pos, x1, y1, x2, y2, winners[e][0],
                         kept[e] & (pos < DETS))
                mark_emitted(winners[e][1], kept[e])
                pos = pos + kept[e].astype(jnp.int32)
            new_cnt = jnp.max(pos)
            wv = []
            for e in range(K):
                x1, y1, x2, y2, ar = coords[e]
                wv.append((jnp.where(kept[e], x1, -1e4),
                           jnp.where(kept[e], y1, -1e4),
                           jnp.where(kept[e], x2, -1e4),
                           jnp.where(kept[e], y2, -1e4),
                           jnp.where(kept[e], ar, 0.0)))
            nst = topk_init()
            for j in range(NV):
                sl = pl.ds(j * 16, 16)
                v = live_v[sl]
                x1 = boxes_v[0, sl]
                y1 = boxes_v[1, sl]
                x2 = boxes_v[2, sl]
                y2 = boxes_v[3, sl]
                ar = area_v[sl]
                sup = jnp.full((16,), False)
                for e in range(K):
                    sup = sup | overlap_vec(*wv[e], x1, y1, x2, y2, ar)
                nv = jnp.where(sup, 0.0, v)
                live_v[sl] = nv
                nst = topk_insert(nv, base + j * 16 + io, nst)
            flat = sum(([v, i] for v, i in nst), [])
            return (new_cnt, flat)

        def fill_branch(_):
            fm = jnp.full((16,), FNEG, jnp.float32)
            fi = jnp.full((16,), 0, jnp.int32)
            for j in range(NV):
                sl = pl.ds(j * 16, 16)
                v = fill_v[sl]
                g = base + j * 16 + io
                upd = v > fm
                fm, fi = jnp.where(upd, v, fm), jnp.where(upd, g, fi)
            smax = jnp.max(fm)
            sidx = jnp.min(jnp.where(fm == smax, fi, IBIG))
            rec_v[pl.ds(0, 16)] = make_rec(smax, sidx)
            pltpu.sync_copy(rec_v.at[pl.ds(0, 16)], recs_s.at[2, tid, pl.ds(0, 16)])
            plsc.subcore_barrier()
            pltpu.sync_copy(recs_s.at[2], recs_v)
            sv0 = col(0, 0)
            ivi0 = col(0, 1).astype(jnp.int32)
            gm = jnp.max(sv0)
            key = jnp.min(jnp.where(sv0 == gm, ivi0 * TCW + io * K, IBIG))
            gi = lax.shift_right_logical(key, 4 + KB)
            tc = key & (TCW - 1)
            x1, y1, x2, y2, _ = fetch_coords(tc)
            pos = jnp.full((16,), 0, jnp.int32) + cnt
            emit_row(pos, x1, y1, x2, y2, jnp.float32(0.0), pos < DETS)
            mark_emitted(gi, jnp.full((16,), True))
            return (cnt + 1, list(st_flat))

        new_cnt, new_flat = lax.cond(keepable, keep_branch, fill_branch, 0)
        return (new_cnt, rnd + 1, tuple(new_flat))

    def round_cond(carry):
        return carry[0] < DETS

    st0_flat = tuple(x for pair in st0 for x in pair)
    lax.while_loop(round_cond, round_body,
                   (jnp.int32(0), jnp.int32(0), st0_flat))

    @pl.when(tid == 0)
    def _():
        pltpu.sync_copy(out_v, out_hbm)


def _make_nms():
    mesh = plsc.VectorSubcoreMesh(core_axis_name="c", subcore_axis_name="s",
                                  num_cores=1)
    return pl.kernel(
        _nms_body,
        out_type=jax.ShapeDtypeStruct((DETS * 16,), jnp.float32),
        mesh=mesh,
        compiler_params=pltpu.CompilerParams(needs_layout_passes=False,
                                             use_tc_tiling_on_sc=False),
        scratch_types=[
            pltpu.VMEM((4, C), jnp.float32),
            pltpu.VMEM((C,), jnp.float32),
            pltpu.VMEM((C,), jnp.float32),
            pltpu.VMEM((C,), jnp.float32),
            pltpu.VMEM((K * 16,), jnp.float32),
            pltpu.VMEM((NT, K * 16), jnp.float32),
            pltpu.VMEM((DETS * 16,), jnp.float32),
            pltpu.VMEM_SHARED((3, NT, K * 16), jnp.float32),
        ],
    )


_nms = _make_nms()


def kernel(boxes, scores):
    pad = NP - N
    bxt = jnp.transpose(boxes)
    bxt = jnp.pad(bxt, ((0, 0), (0, pad)), constant_values=-1e4)
    sc = jnp.pad(scores.astype(jnp.float32), (0, pad), constant_values=-1.0)
    out = _nms(bxt.reshape(-1).astype(jnp.float32), sc)
    return out.reshape(DETS, 16)[:, :5]


# speedup vs baseline: 1.0923x; 1.0923x over previous
import jax
import jax.numpy as jnp
from jax import lax
from jax.experimental import pallas as pl
from jax.experimental.pallas import tpu as pltpu
from jax.experimental.pallas import tpu_sc as plsc

N = 5000
NT = 16
C = 320
NP = NT * C
NV = C // 16
K = 4
DETS = 100
NMS_THRESH = 0.5
SCORE_THRESH = 0.05
FNEG = -3.4e38
IBIG = 2**31 - 1


def _nms_body(bx_hbm, sc_hbm, out_hbm, boxes_v, area_v, live_v, fill_v,
              rec_v, recs_v, out_v, recs_s):
    tid = lax.axis_index("s")
    base = tid * C
    io = lax.iota(jnp.int32, 16)

    for c in range(4):
        pltpu.sync_copy(bx_hbm.at[pl.ds(c * NP + base, C)], boxes_v.at[c])
    pltpu.sync_copy(sc_hbm.at[pl.ds(base, C)], live_v)
    pltpu.sync_copy(sc_hbm.at[pl.ds(base, C)], fill_v)

    for j in range(NV):
        sl = pl.ds(j * 16, 16)
        area_v[sl] = ((boxes_v[2, sl] - boxes_v[0, sl]) *
                      (boxes_v[3, sl] - boxes_v[1, sl]))

    def top4_insert(v, g, st):
        (m1, i1), (m2, i2), (m3, i3), (m4, i4) = st
        c1, c2 = v > m1, v > m2
        c3, c4 = v > m3, v > m4
        n1 = jnp.where(c1, v, m1)
        j1 = jnp.where(c1, g, i1)
        n2 = jnp.where(c1, m1, jnp.where(c2, v, m2))
        j2 = jnp.where(c1, i1, jnp.where(c2, g, i2))
        n3 = jnp.where(c2, m2, jnp.where(c3, v, m3))
        j3 = jnp.where(c2, i2, jnp.where(c3, g, i3))
        n4 = jnp.where(c3, m3, jnp.where(c4, v, m4))
        j4 = jnp.where(c3, i3, jnp.where(c4, g, i4))
        return ((n1, j1), (n2, j2), (n3, j3), (n4, j4))

    def top4_init():
        return tuple((jnp.full((16,), FNEG, jnp.float32),
                      jnp.full((16,), 0, jnp.int32)) for _ in range(K))

    def init_pass():
        st = top4_init()
        for j in range(NV):
            sl = pl.ds(j * 16, 16)
            st = top4_insert(live_v[sl], base + j * 16 + io, st)
        return st

    def tile_extract(st):
        vals = [v for v, _ in st]
        idxs = [i for _, i in st]
        out = []
        for _ in range(K):
            gm = jnp.max(jnp.maximum(jnp.maximum(vals[0], vals[1]),
                                     jnp.maximum(vals[2], vals[3])))
            gi = jnp.min(jnp.minimum(
                jnp.minimum(jnp.where(vals[0] == gm, idxs[0], IBIG),
                            jnp.where(vals[1] == gm, idxs[1], IBIG)),
                jnp.minimum(jnp.where(vals[2] == gm, idxs[2], IBIG),
                            jnp.where(vals[3] == gm, idxs[3], IBIG))))
            out.append((gm, gi))
            vals = [jnp.where(idxs[k] == gi, FNEG, vals[k]) for k in range(K)]
        return out

    def make_rec(s, gidx):
        lidx = jnp.clip(gidx - base, 0, C - 1)
        cvec = plsc.load_gather(boxes_v, [jnp.clip(io - 2, 0, 3),
                                          jnp.full((16,), lidx, jnp.int32)])
        avec = plsc.load_gather(area_v, [jnp.full((16,), lidx, jnp.int32)])
        return jnp.where(io == 0, s,
               jnp.where(io == 1, gidx.astype(jnp.float32),
               jnp.where(io == 6, avec, cvec)))

    def publish(slot, cands):
        for e, (s, gi) in enumerate(cands):
            rec_v[pl.ds(e * 16, 16)] = make_rec(s, gi)
        pltpu.sync_copy(rec_v, recs_s.at[slot, tid])
        plsc.subcore_barrier()
        pltpu.sync_copy(recs_s.at[slot], recs_v)
        pltpu.sync_copy(recs_s.at[slot], recs_v)

    def col(c, f):
        return plsc.load_gather(recs_v, [io, jnp.full((16,), c * 16 + f,
                                                      jnp.int32)])

    def splat_field(t_e, off_e):
        return plsc.load_gather(recs_v, [jnp.full((16,), t_e, jnp.int32),
                                         jnp.full((16,), off_e, jnp.int32)])

    def global_extract():
        sv = [col(c, 0) for c in range(K)]
        ivi = [col(c, 1).astype(jnp.int32) for c in range(K)]
        winners = []
        for _ in range(K):
            gm = jnp.max(jnp.maximum(jnp.maximum(sv[0], sv[1]),
                                     jnp.maximum(sv[2], sv[3])))
            key = jnp.min(jnp.minimum(
                jnp.minimum(
                    jnp.where(sv[0] == gm, ivi[0] * 64 + io * 4 + 0, IBIG),
                    jnp.where(sv[1] == gm, ivi[1] * 64 + io * 4 + 1, IBIG)),
                jnp.minimum(
                    jnp.where(sv[2] == gm, ivi[2] * 64 + io * 4 + 2, IBIG),
                    jnp.where(sv[3] == gm, ivi[3] * 64 + io * 4 + 3, IBIG))))
            gi = lax.shift_right_logical(key, 6)
            tc = key & 63
            winners.append((gm, gi, tc))
            sv = [jnp.where(ivi[c] == gi, FNEG, sv[c]) for c in range(K)]
        return winners

    def fetch_coords(tc):
        t_e = lax.shift_right_logical(tc, 2)
        o_e = (tc & 3) * 16
        return (splat_field(t_e, o_e + 2), splat_field(t_e, o_e + 3),
                splat_field(t_e, o_e + 4), splat_field(t_e, o_e + 5),
                splat_field(t_e, o_e + 6))

    def overlap_vec(ax1, ay1, ax2, ay2, aar, bx1, by1, bx2, by2, bar):
        w = jnp.maximum(jnp.minimum(ax2, bx2) - jnp.maximum(ax1, bx1), 0.0)
        h = jnp.maximum(jnp.minimum(ay2, by2) - jnp.maximum(ay1, by1), 0.0)
        inter = w * h
        union = (aar + bar) - inter
        return (inter + inter) > union

    def emit_row(pos_vec, x1, y1, x2, y2, s, mask):
        row = jnp.where(io == 0, x1,
              jnp.where(io == 1, y1,
              jnp.where(io == 2, x2,
              jnp.where(io == 3, y2,
              jnp.where(io == 4, s, 0.0)))))
        posc = jnp.minimum(pos_vec, DETS - 1)
        @pl.when(tid == 0)
        def _():
            plsc.store_scatter(out_v, [posc * 16 + io], row, mask=mask)

    def mark_emitted(gidx, keep_mask):
        lidx = gidx.astype(jnp.int32) - base
        inr = (lidx >= 0) & (lidx < C)
        idxv = jnp.full((16,), jnp.clip(lidx, 0, C - 1), jnp.int32)
        plsc.store_scatter(fill_v, [idxv], jnp.full((16,), -2.0, jnp.float32),
                           mask=(io == 0) & inr & keep_mask)

    st0 = init_pass()

    def round_body(carry):
        cnt, rnd, st_flat = carry
        st = tuple((st_flat[2 * k], st_flat[2 * k + 1]) for k in range(K))
        slot = lax.rem(rnd, 2)
        publish(slot, tile_extract(st))
        winners = global_extract()
        keepable = winners[0][0] > SCORE_THRESH

        def keep_branch(_):
            coords = [fetch_coords(tc) for (_, _, tc) in winners]
            kept = []
            for e in range(K):
                s_e = winners[e][0]
                ok = jnp.full((16,), True)
                for i in range(e):
                    ov = overlap_vec(*coords[i], *coords[e])
                    ok = ok & ~(kept[i] & ov)
                kept.append(ok & (s_e > SCORE_THRESH))
            pos = jnp.full((16,), 0, jnp.int32) + cnt
            for e in range(K):
                x1, y1, x2, y2, _ = coords[e]
                emit_row(pos, x1, y1, x2, y2, winners[e][0],
                         kept[e] & (pos < DETS))
                mark_emitted(winners[e][1], kept[e])
                pos = pos + kept[e].astype(jnp.int32)
            new_cnt = jnp.max(pos)
            wv = []
            for e in range(K):
                x1, y1, x2, y2, ar = coords[e]
                wv.append((jnp.where(kept[e], x1, -1e4),
                           jnp.where(kept[e], y1, -1e4),
                           jnp.where(kept[e], x2, -1e4),
                           jnp.where(kept[e], y2, -1e4),
                           jnp.where(kept[e], ar, 0.0)))
            nst = top4_init()
            for j in range(NV):
                sl = pl.ds(j * 16, 16)
                v = live_v[sl]
                x1 = boxes_v[0, sl]
                y1 = boxes_v[1, sl]
                x2 = boxes_v[2, sl]
                y2 = boxes_v[3, sl]
                ar = area_v[sl]
                sup = jnp.full((16,), False)
                for e in range(K):
                    sup = sup | overlap_vec(*wv[e], x1, y1, x2, y2, ar)
                nv = jnp.where(sup, 0.0, v)
                live_v[sl] = nv
                nst = top4_insert(nv, base + j * 16 + io, nst)
            flat = sum(([v, i] for v, i in nst), [])
            return (new_cnt, flat)

        def fill_branch(_):
            fm = jnp.full((16,), FNEG, jnp.float32)
            fi = jnp.full((16,), 0, jnp.int32)
            for j in range(NV):
                sl = pl.ds(j * 16, 16)
                v = fill_v[sl]
                g = base + j * 16 + io
                upd = v > fm
                fm, fi = jnp.where(upd, v, fm), jnp.where(upd, g, fi)
            smax = jnp.max(fm)
            sidx = jnp.min(jnp.where(fm == smax, fi, IBIG))
            rec_v[pl.ds(0, 16)] = make_rec(smax, sidx)
            pltpu.sync_copy(rec_v.at[pl.ds(0, 16)], recs_s.at[2, tid, pl.ds(0, 16)])
            plsc.subcore_barrier()
            pltpu.sync_copy(recs_s.at[2], recs_v)
            sv0 = col(0, 0)
            ivi0 = col(0, 1).astype(jnp.int32)
            gm = jnp.max(sv0)
            key = jnp.min(jnp.where(sv0 == gm, ivi0 * 64 + io * 4, IBIG))
            gi = lax.shift_right_logical(key, 6)
            tc = key & 63
            x1, y1, x2, y2, _ = fetch_coords(tc)
            pos = jnp.full((16,), 0, jnp.int32) + cnt
            emit_row(pos, x1, y1, x2, y2, jnp.float32(0.0), pos < DETS)
            mark_emitted(gi, jnp.full((16,), True))
            return (cnt + 1, list(st_flat))

        new_cnt, new_flat = lax.cond(keepable, keep_branch, fill_branch, 0)
        return (new_cnt, rnd + 1, tuple(new_flat))

    def round_cond(carry):
        return carry[0] < DETS

    st0_flat = tuple(x for pair in st0 for x in pair)
    lax.while_loop(round_cond, round_body,
                   (jnp.int32(0), jnp.int32(0), st0_flat))

    @pl.when(tid == 0)
    def _():
        pltpu.sync_copy(out_v, out_hbm)


def _make_nms():
    mesh = plsc.VectorSubcoreMesh(core_axis_name="c", subcore_axis_name="s",
                                  num_cores=1)
    return pl.kernel(
        _nms_body,
        out_type=jax.ShapeDtypeStruct((DETS * 16,), jnp.float32),
        mesh=mesh,
        compiler_params=pltpu.CompilerParams(needs_layout_passes=False,
                                             use_tc_tiling_on_sc=False),
        scratch_types=[
            pltpu.VMEM((4, C), jnp.float32),
            pltpu.VMEM((C,), jnp.float32),
            pltpu.VMEM((C,), jnp.float32),
            pltpu.VMEM((C,), jnp.float32),
            pltpu.VMEM((K * 16,), jnp.float32),
            pltpu.VMEM((NT, K * 16), jnp.float32),
            pltpu.VMEM((DETS * 16,), jnp.float32),
            pltpu.VMEM_SHARED((3, NT, K * 16), jnp.float32),
        ],
    )


_nms = _make_nms()


def kernel(boxes, scores):
    pad = NP - N
    bxt = jnp.transpose(boxes)
    bxt = jnp.pad(bxt, ((0, 0), (0, pad)), constant_values=-1e4)
    sc = jnp.pad(scores.astype(jnp.float32), (0, pad), constant_values=-1.0)
    out = _nms(bxt.reshape(-1).astype(jnp.float32), sc)
    return out.reshape(DETS, 16)[:, :5]


# speedup vs baseline: 1.1882x; 1.0877x over previous
import jax
import jax.numpy as jnp
from jax import lax
from jax.experimental import pallas as pl
from jax.experimental.pallas import tpu as pltpu
from jax.experimental.pallas import tpu_sc as plsc

N = 5000
NT = 16
C = 320
NP = NT * C
NV = C // 16
K = 4
DETS = 100
NMS_THRESH = 0.5
SCORE_THRESH = 0.05
FNEG = -3.4e38
IBIG = 2**31 - 1


def _nms_body(bx_hbm, sc_hbm, out_hbm, bxr_v, boxes_v, area_v, live_v, fill_v,
              rec_v, recs_v, out_v, recs_s):
    tid = lax.axis_index("s")
    base = tid * C
    io = lax.iota(jnp.int32, 16)

    NLAST = N - (NT - 1) * C

    @pl.when(tid < NT - 1)
    def _():
        pltpu.sync_copy(bx_hbm.at[pl.ds(base * 4, C * 4)], bxr_v)
        pltpu.sync_copy(sc_hbm.at[pl.ds(base, C)], live_v)

    @pl.when(tid == NT - 1)
    def _():
        pltpu.sync_copy(bx_hbm.at[pl.ds((NT - 1) * C * 4, NLAST * 4)],
                        bxr_v.at[pl.ds(0, NLAST * 4)])
        pltpu.sync_copy(sc_hbm.at[pl.ds((NT - 1) * C, NLAST)],
                        live_v.at[pl.ds(0, NLAST)])

    io4 = io * 4
    for j in range(NV):
        sl = pl.ds(j * 16, 16)
        padm = (base + j * 16 + io) >= N
        cols = []
        for c in range(4):
            raw = plsc.load_gather(bxr_v, [io4 + (j * 64 + c)])
            cols.append(jnp.where(padm, -1e4, raw))
        for c in range(4):
            boxes_v[c, sl] = cols[c]
        area_v[sl] = (cols[2] - cols[0]) * (cols[3] - cols[1])
        v = jnp.where(padm, -1.0, live_v[sl])
        live_v[sl] = v
        fill_v[sl] = v

    def top4_insert(v, g, st):
        (m1, i1), (m2, i2), (m3, i3), (m4, i4) = st
        c1, c2 = v > m1, v > m2
        c3, c4 = v > m3, v > m4
        n1 = jnp.where(c1, v, m1)
        j1 = jnp.where(c1, g, i1)
        n2 = jnp.where(c1, m1, jnp.where(c2, v, m2))
        j2 = jnp.where(c1, i1, jnp.where(c2, g, i2))
        n3 = jnp.where(c2, m2, jnp.where(c3, v, m3))
        j3 = jnp.where(c2, i2, jnp.where(c3, g, i3))
        n4 = jnp.where(c3, m3, jnp.where(c4, v, m4))
        j4 = jnp.where(c3, i3, jnp.where(c4, g, i4))
        return ((n1, j1), (n2, j2), (n3, j3), (n4, j4))

    def top4_init():
        return tuple((jnp.full((16,), FNEG, jnp.float32),
                      jnp.full((16,), 0, jnp.int32)) for _ in range(K))

    def init_pass():
        st = top4_init()
        for j in range(NV):
            sl = pl.ds(j * 16, 16)
            st = top4_insert(live_v[sl], base + j * 16 + io, st)
        return st

    def tile_extract(st):
        vals = [v for v, _ in st]
        idxs = [i for _, i in st]
        out = []
        for _ in range(K):
            gm = jnp.max(jnp.maximum(jnp.maximum(vals[0], vals[1]),
                                     jnp.maximum(vals[2], vals[3])))
            gi = jnp.min(jnp.minimum(
                jnp.minimum(jnp.where(vals[0] == gm, idxs[0], IBIG),
                            jnp.where(vals[1] == gm, idxs[1], IBIG)),
                jnp.minimum(jnp.where(vals[2] == gm, idxs[2], IBIG),
                            jnp.where(vals[3] == gm, idxs[3], IBIG))))
            out.append((gm, gi))
            vals = [jnp.where(idxs[k] == gi, FNEG, vals[k]) for k in range(K)]
        return out

    def make_rec(s, gidx):
        lidx = jnp.clip(gidx - base, 0, C - 1)
        cvec = plsc.load_gather(boxes_v, [jnp.clip(io - 2, 0, 3),
                                          jnp.full((16,), lidx, jnp.int32)])
        avec = plsc.load_gather(area_v, [jnp.full((16,), lidx, jnp.int32)])
        return jnp.where(io == 0, s,
               jnp.where(io == 1, gidx.astype(jnp.float32),
               jnp.where(io == 6, avec, cvec)))

    def publish(slot, cands):
        for e, (s, gi) in enumerate(cands):
            rec_v[pl.ds(e * 16, 16)] = make_rec(s, gi)
        pltpu.sync_copy(rec_v, recs_s.at[slot, tid])
        plsc.subcore_barrier()
        pltpu.sync_copy(recs_s.at[slot], recs_v)

    def col(c, f):
        return plsc.load_gather(recs_v, [io, jnp.full((16,), c * 16 + f,
                                                      jnp.int32)])

    def splat_field(t_e, off_e):
        return plsc.load_gather(recs_v, [jnp.full((16,), t_e, jnp.int32),
                                         jnp.full((16,), off_e, jnp.int32)])

    def global_extract():
        sv = [col(c, 0) for c in range(K)]
        ivi = [col(c, 1).astype(jnp.int32) for c in range(K)]
        winners = []
        for _ in range(K):
            gm = jnp.max(jnp.maximum(jnp.maximum(sv[0], sv[1]),
                                     jnp.maximum(sv[2], sv[3])))
            key = jnp.min(jnp.minimum(
                jnp.minimum(
                    jnp.where(sv[0] == gm, ivi[0] * 64 + io * 4 + 0, IBIG),
                    jnp.where(sv[1] == gm, ivi[1] * 64 + io * 4 + 1, IBIG)),
                jnp.minimum(
                    jnp.where(sv[2] == gm, ivi[2] * 64 + io * 4 + 2, IBIG),
                    jnp.where(sv[3] == gm, ivi[3] * 64 + io * 4 + 3, IBIG))))
            gi = lax.shift_right_logical(key, 6)
            tc = key & 63
            winners.append((gm, gi, tc))
            sv = [jnp.where(ivi[c] == gi, FNEG, sv[c]) for c in range(K)]
        return winners

    def fetch_coords(tc):
        t_e = lax.shift_right_logical(tc, 2)
        o_e = (tc & 3) * 16
        return (splat_field(t_e, o_e + 2), splat_field(t_e, o_e + 3),
                splat_field(t_e, o_e + 4), splat_field(t_e, o_e + 5),
                splat_field(t_e, o_e + 6))

    def overlap_vec(ax1, ay1, ax2, ay2, aar, bx1, by1, bx2, by2, bar):
        w = jnp.maximum(jnp.minimum(ax2, bx2) - jnp.maximum(ax1, bx1), 0.0)
        h = jnp.maximum(jnp.minimum(ay2, by2) - jnp.maximum(ay1, by1), 0.0)
        inter = w * h
        union = (aar + bar) - inter
        return (inter + inter) > union

    def emit_row(pos_vec, x1, y1, x2, y2, s, mask):
        row = jnp.where(io == 0, x1,
              jnp.where(io == 1, y1,
              jnp.where(io == 2, x2,
              jnp.where(io == 3, y2,
              jnp.where(io == 4, s, 0.0)))))
        posc = jnp.minimum(pos_vec, DETS - 1)
        @pl.when(tid == 0)
        def _():
            plsc.store_scatter(out_v, [posc * 5 + io], row,
                               mask=mask & (io < 5))

    def mark_emitted(gidx, keep_mask):
        lidx = gidx.astype(jnp.int32) - base
        inr = (lidx >= 0) & (lidx < C)
        idxv = jnp.full((16,), jnp.clip(lidx, 0, C - 1), jnp.int32)
        plsc.store_scatter(fill_v, [idxv], jnp.full((16,), -2.0, jnp.float32),
                           mask=(io == 0) & inr & keep_mask)

    st0 = init_pass()

    def round_body(carry):
        cnt, rnd, st_flat = carry
        st = tuple((st_flat[2 * k], st_flat[2 * k + 1]) for k in range(K))
        slot = lax.rem(rnd, 2)
        publish(slot, tile_extract(st))
        winners = global_extract()
        keepable = winners[0][0] > SCORE_THRESH

        def keep_branch(_):
            coords = [fetch_coords(tc) for (_, _, tc) in winners]
            kept = []
            for e in range(K):
                s_e = winners[e][0]
                ok = jnp.full((16,), True)
                for i in range(e):
                    ov = overlap_vec(*coords[i], *coords[e])
                    ok = ok & ~(kept[i] & ov)
                kept.append(ok & (s_e > SCORE_THRESH))
            pos = jnp.full((16,), 0, jnp.int32) + cnt
            for e in range(K):
                x1, y1, x2, y2, _ = coords[e]
                emit_row(pos, x1, y1, x2, y2, winners[e][0],
                         kept[e] & (pos < DETS))
                mark_emitted(winners[e][1], kept[e])
                pos = pos + kept[e].astype(jnp.int32)
            new_cnt = jnp.max(pos)
            wv = []
            for e in range(K):
                x1, y1, x2, y2, ar = coords[e]
                wv.append((jnp.where(kept[e], x1, -1e4),
                           jnp.where(kept[e], y1, -1e4),
                           jnp.where(kept[e], x2, -1e4),
                           jnp.where(kept[e], y2, -1e4),
                           jnp.where(kept[e], ar, 0.0)))
            nst = top4_init()
            for j in range(NV):
                sl = pl.ds(j * 16, 16)
                v = live_v[sl]
                x1 = boxes_v[0, sl]
                y1 = boxes_v[1, sl]
                x2 = boxes_v[2, sl]
                y2 = boxes_v[3, sl]
                ar = area_v[sl]
                sup = jnp.full((16,), False)
                for e in range(K):
                    sup = sup | overlap_vec(*wv[e], x1, y1, x2, y2, ar)
                nv = jnp.where(sup, 0.0, v)
                live_v[sl] = nv
                nst = top4_insert(nv, base + j * 16 + io, nst)
            flat = sum(([v, i] for v, i in nst), [])
            return (new_cnt, flat)

        def fill_branch(_):
            fm = jnp.full((16,), FNEG, jnp.float32)
            fi = jnp.full((16,), 0, jnp.int32)
            for j in range(NV):
                sl = pl.ds(j * 16, 16)
                v = fill_v[sl]
                g = base + j * 16 + io
                upd = v > fm
                fm, fi = jnp.where(upd, v, fm), jnp.where(upd, g, fi)
            smax = jnp.max(fm)
            sidx = jnp.min(jnp.where(fm == smax, fi, IBIG))
            rec_v[pl.ds(0, 16)] = make_rec(smax, sidx)
            pltpu.sync_copy(rec_v.at[pl.ds(0, 16)], recs_s.at[2, tid, pl.ds(0, 16)])
            plsc.subcore_barrier()
            pltpu.sync_copy(recs_s.at[2], recs_v)
            sv0 = col(0, 0)
            ivi0 = col(0, 1).astype(jnp.int32)
            gm = jnp.max(sv0)
            key = jnp.min(jnp.where(sv0 == gm, ivi0 * 64 + io * 4, IBIG))
            gi = lax.shift_right_logical(key, 6)
            tc = key & 63
            x1, y1, x2, y2, _ = fetch_coords(tc)
            pos = jnp.full((16,), 0, jnp.int32) + cnt
            emit_row(pos, x1, y1, x2, y2, jnp.float32(0.0), pos < DETS)
            mark_emitted(gi, jnp.full((16,), True))
            return (cnt + 1, list(st_flat))

        new_cnt, new_flat = lax.cond(keepable, keep_branch, fill_branch, 0)
        return (new_cnt, rnd + 1, tuple(new_flat))

    def round_cond(carry):
        return carry[0] < DETS

    st0_flat = tuple(x for pair in st0 for x in pair)
    lax.while_loop(round_cond, round_body,
                   (jnp.int32(0), jnp.int32(0), st0_flat))

    @pl.when(tid == 0)
    def _():
        pltpu.sync_copy(out_v.at[pl.ds(0, DETS * 5)], out_hbm)


def _make_nms():
    mesh = plsc.VectorSubcoreMesh(core_axis_name="c", subcore_axis_name="s",
                                  num_cores=1)
    return pl.kernel(
        _nms_body,
        out_type=jax.ShapeDtypeStruct((DETS * 5,), jnp.float32),
        mesh=mesh,
        compiler_params=pltpu.CompilerParams(needs_layout_passes=False,
                                             use_tc_tiling_on_sc=False),
        scratch_types=[
            pltpu.VMEM((4 * C,), jnp.float32),
            pltpu.VMEM((4, C), jnp.float32),
            pltpu.VMEM((C,), jnp.float32),
            pltpu.VMEM((C,), jnp.float32),
            pltpu.VMEM((C,), jnp.float32),
            pltpu.VMEM((K * 16,), jnp.float32),
            pltpu.VMEM((NT, K * 16), jnp.float32),
            pltpu.VMEM((DETS * 5 + 12,), jnp.float32),
            pltpu.VMEM_SHARED((3, NT, K * 16), jnp.float32),
        ],
    )


_nms = _make_nms()


def kernel(boxes, scores):
    out = _nms(boxes.astype(jnp.float32).reshape(-1),
               scores.astype(jnp.float32))
    return out.reshape(DETS, 5)
